# baseline (speedup 1.0000x reference)
import functools
import math

import jax
import jax.numpy as jnp
from jax import lax
from jax.experimental import pallas as pl
from jax.experimental.pallas import tpu as pltpu
from jax.experimental.pallas import tpu_sc as plsc

BATCH = 4
SEQ = 4096
DIM = 1024
NUM_ROWS = BATCH * SEQ
NC, NS, LANES = 2, 16, 16
NW = NC * NS
RPW = NUM_ROWS // NW
WPB = SEQ // RPW
CHUNK = 16
NCHUNK = RPW // CHUNK
NBUF = 7
LEAD = 2
SCALE = math.sqrt(DIM)


def _sc_body(x_hbm, table_hbm, out_hbm, idx_v,
             b0, b1, b2, b3, b4, b5, b6,
             sg0, sg1, sg2, sg3, sg4, sg5, sg6,
             so0, so1, so2, so3, so4, so5, so6):
    bufs = (b0, b1, b2, b3, b4, b5, b6)
    sgs = (sg0, sg1, sg2, sg3, sg4, sg5, sg6)
    sos = (so0, so1, so2, so3, so4, so5, so6)
    wid = lax.axis_index("s") * NC + lax.axis_index("c")
    batch = wid // WPB
    col0 = (wid % WPB) * RPW
    pltpu.sync_copy(x_hbm.at[batch, pl.ds(col0, RPW)], idx_v)

    def gather(g):
        k = g % NBUF
        return pltpu.async_copy(
            table_hbm.at[idx_v.at[pl.ds(g * CHUNK, CHUNK)]], bufs[k], sgs[k])

    def writeback(g):
        k = g % NBUF
        return pltpu.async_copy(
            bufs[k], out_hbm.at[batch, pl.ds(col0 + g * CHUNK, CHUNK)], sos[k])

    def scale(k):
        def half_body(i, c2):
            r = i >> 1
            cb = (i & 1) * (DIM // 2)
            for c in range(DIM // LANES // 2):
                sl = pl.ds(cb + c * LANES, LANES)
                bufs[k][r, sl] = bufs[k][r, sl] * SCALE
            return c2
        lax.fori_loop(0, 2 * CHUNK, half_body, 0, unroll=False)

    hg = {g: gather(g) for g in range(LEAD)}
    hw = {}
    for g in range(NCHUNK):
        k = g % NBUF
        hg[g].wait()
        scale(k)
        hw[g] = writeback(g)
        if g + LEAD < NCHUNK:
            prev = g + LEAD - NBUF
            if prev >= 0:
                hw[prev].wait()
            hg[g + LEAD] = gather(g + LEAD)
    for g in range(NCHUNK - NBUF, NCHUNK):
        hw[g].wait()


@jax.jit
def kernel(x, table):
    mesh = plsc.VectorSubcoreMesh(core_axis_name="c", subcore_axis_name="s")
    f = functools.partial(
        pl.kernel,
        out_type=jax.ShapeDtypeStruct((BATCH, SEQ, DIM), jnp.float32),
        mesh=mesh,
        scratch_types=(
            [pltpu.VMEM((RPW,), jnp.int32)]
            + [pltpu.VMEM((CHUNK, DIM), jnp.float32)] * NBUF
            + [pltpu.SemaphoreType.DMA] * (2 * NBUF)
        ),
    )(_sc_body)
    return f(x, table)


# speedup vs baseline: 1.0203x; 1.0203x over previous
import functools
import math

import jax
import jax.numpy as jnp
from jax import lax
from jax.experimental import pallas as pl
from jax.experimental.pallas import tpu as pltpu
from jax.experimental.pallas import tpu_sc as plsc

BATCH = 4
SEQ = 4096
DIM = 1024
NUM_ROWS = BATCH * SEQ
NC, NS, LANES = 2, 16, 16
NW = NC * NS
RPW = NUM_ROWS // NW
WPB = SEQ // RPW
CHUNK = 16
NCHUNK = RPW // CHUNK
NBUF = 7
LEAD = 4
SCALE = math.sqrt(DIM)


def _sc_body(x_hbm, table_hbm, out_hbm, idx_v,
             b0, b1, b2, b3, b4, b5, b6,
             sg0, sg1, sg2, sg3, sg4, sg5, sg6,
             so0, so1, so2, so3, so4, so5, so6):
    bufs = (b0, b1, b2, b3, b4, b5, b6)
    sgs = (sg0, sg1, sg2, sg3, sg4, sg5, sg6)
    sos = (so0, so1, so2, so3, so4, so5, so6)
    wid = lax.axis_index("s") * NC + lax.axis_index("c")
    batch = wid // WPB
    col0 = (wid % WPB) * RPW
    pltpu.sync_copy(x_hbm.at[batch, pl.ds(col0, RPW)], idx_v)

    def gather(g):
        k = g % NBUF
        return pltpu.async_copy(
            table_hbm.at[idx_v.at[pl.ds(g * CHUNK, CHUNK)]], bufs[k], sgs[k])

    def writeback(g):
        k = g % NBUF
        return pltpu.async_copy(
            bufs[k], out_hbm.at[batch, pl.ds(col0 + g * CHUNK, CHUNK)], sos[k])

    def scale(k):
        def half_body(i, c2):
            r = i >> 1
            cb = (i & 1) * (DIM // 2)
            for c in range(DIM // LANES // 2):
                sl = pl.ds(cb + c * LANES, LANES)
                bufs[k][r, sl] = bufs[k][r, sl] * SCALE
            return c2
        lax.fori_loop(0, 2 * CHUNK, half_body, 0, unroll=False)

    hg = {g: gather(g) for g in range(LEAD)}
    hw = {}
    for g in range(NCHUNK):
        k = g % NBUF
        hg[g].wait()
        scale(k)
        hw[g] = writeback(g)
        if g + LEAD < NCHUNK:
            prev = g + LEAD - NBUF
            if prev >= 0:
                hw[prev].wait()
            hg[g + LEAD] = gather(g + LEAD)
    for g in range(NCHUNK - NBUF, NCHUNK):
        hw[g].wait()


@jax.jit
def kernel(x, table):
    mesh = plsc.VectorSubcoreMesh(core_axis_name="c", subcore_axis_name="s")
    f = functools.partial(
        pl.kernel,
        out_type=jax.ShapeDtypeStruct((BATCH, SEQ, DIM), jnp.float32),
        mesh=mesh,
        scratch_types=(
            [pltpu.VMEM((RPW,), jnp.int32)]
            + [pltpu.VMEM((CHUNK, DIM), jnp.float32)] * NBUF
            + [pltpu.SemaphoreType.DMA] * (2 * NBUF)
        ),
    )(_sc_body)
    return f(x, table)


# speedup vs baseline: 1.0362x; 1.0157x over previous
import functools
import math

import jax
import jax.numpy as jnp
from jax import lax
from jax.experimental import pallas as pl
from jax.experimental.pallas import tpu as pltpu
from jax.experimental.pallas import tpu_sc as plsc

BATCH = 4
SEQ = 4096
DIM = 1024
NUM_ROWS = BATCH * SEQ
NC, NS, LANES = 2, 16, 16
NW = NC * NS
RPW = NUM_ROWS // NW
WPB = SEQ // RPW
CHUNK = 16
NCHUNK = RPW // CHUNK
NBUF = 7
LEAD = 5
SCALE = math.sqrt(DIM)


def _sc_body(x_hbm, table_hbm, out_hbm, idx_v,
             b0, b1, b2, b3, b4, b5, b6,
             sg0, sg1, sg2, sg3, sg4, sg5, sg6,
             so0, so1, so2, so3, so4, so5, so6):
    bufs = (b0, b1, b2, b3, b4, b5, b6)
    sgs = (sg0, sg1, sg2, sg3, sg4, sg5, sg6)
    sos = (so0, so1, so2, so3, so4, so5, so6)
    wid = lax.axis_index("s") * NC + lax.axis_index("c")
    batch = wid // WPB
    col0 = (wid % WPB) * RPW
    pltpu.sync_copy(x_hbm.at[batch, pl.ds(col0, RPW)], idx_v)

    def gather(g):
        k = g % NBUF
        return pltpu.async_copy(
            table_hbm.at[idx_v.at[pl.ds(g * CHUNK, CHUNK)]], bufs[k], sgs[k])

    def writeback(g):
        k = g % NBUF
        return pltpu.async_copy(
            bufs[k], out_hbm.at[batch, pl.ds(col0 + g * CHUNK, CHUNK)], sos[k])

    def scale(k):
        def half_body(i, c2):
            r = i >> 1
            cb = (i & 1) * (DIM // 2)
            for c in range(DIM // LANES // 2):
                sl = pl.ds(cb + c * LANES, LANES)
                bufs[k][r, sl] = bufs[k][r, sl] * SCALE
            return c2
        lax.fori_loop(0, 2 * CHUNK, half_body, 0, unroll=False)

    hg = {g: gather(g) for g in range(LEAD)}
    hw = {}
    for g in range(NCHUNK):
        k = g % NBUF
        hg[g].wait()
        scale(k)
        hw[g] = writeback(g)
        if g + LEAD < NCHUNK:
            prev = g + LEAD - NBUF
            if prev >= 0:
                hw[prev].wait()
            hg[g + LEAD] = gather(g + LEAD)
    for g in range(NCHUNK - NBUF, NCHUNK):
        hw[g].wait()


@jax.jit
def kernel(x, table):
    mesh = plsc.VectorSubcoreMesh(core_axis_name="c", subcore_axis_name="s")
    f = functools.partial(
        pl.kernel,
        out_type=jax.ShapeDtypeStruct((BATCH, SEQ, DIM), jnp.float32),
        mesh=mesh,
        scratch_types=(
            [pltpu.VMEM((RPW,), jnp.int32)]
            + [pltpu.VMEM((CHUNK, DIM), jnp.float32)] * NBUF
            + [pltpu.SemaphoreType.DMA] * (2 * NBUF)
        ),
    )(_sc_body)
    return f(x, table)


# speedup vs baseline: 1.0473x; 1.0106x over previous
import functools
import math

import jax
import jax.numpy as jnp
from jax import lax
from jax.experimental import pallas as pl
from jax.experimental.pallas import tpu as pltpu
from jax.experimental.pallas import tpu_sc as plsc

BATCH = 4
SEQ = 4096
DIM = 1024
NUM_ROWS = BATCH * SEQ
NC, NS, LANES = 2, 16, 16
NW = NC * NS
RPW = NUM_ROWS // NW
WPB = SEQ // RPW
CHUNK = 16
NCHUNK = RPW // CHUNK
NBUF = 7
LEAD = 6
SCALE = math.sqrt(DIM)


def _sc_body(x_hbm, table_hbm, out_hbm, idx_v,
             b0, b1, b2, b3, b4, b5, b6,
             sg0, sg1, sg2, sg3, sg4, sg5, sg6,
             so0, so1, so2, so3, so4, so5, so6):
    bufs = (b0, b1, b2, b3, b4, b5, b6)
    sgs = (sg0, sg1, sg2, sg3, sg4, sg5, sg6)
    sos = (so0, so1, so2, so3, so4, so5, so6)
    wid = lax.axis_index("s") * NC + lax.axis_index("c")
    batch = wid // WPB
    col0 = (wid % WPB) * RPW
    pltpu.sync_copy(x_hbm.at[batch, pl.ds(col0, RPW)], idx_v)

    def gather(g):
        k = g % NBUF
        return pltpu.async_copy(
            table_hbm.at[idx_v.at[pl.ds(g * CHUNK, CHUNK)]], bufs[k], sgs[k])

    def writeback(g):
        k = g % NBUF
        return pltpu.async_copy(
            bufs[k], out_hbm.at[batch, pl.ds(col0 + g * CHUNK, CHUNK)], sos[k])

    def scale(k):
        def half_body(i, c2):
            r = i >> 1
            cb = (i & 1) * (DIM // 2)
            for c in range(DIM // LANES // 2):
                sl = pl.ds(cb + c * LANES, LANES)
                bufs[k][r, sl] = bufs[k][r, sl] * SCALE
            return c2
        lax.fori_loop(0, 2 * CHUNK, half_body, 0, unroll=False)

    hg = {g: gather(g) for g in range(LEAD)}
    hw = {}
    for g in range(NCHUNK):
        k = g % NBUF
        hg[g].wait()
        scale(k)
        hw[g] = writeback(g)
        if g + LEAD < NCHUNK:
            prev = g + LEAD - NBUF
            if prev >= 0:
                hw[prev].wait()
            hg[g + LEAD] = gather(g + LEAD)
    for g in range(NCHUNK - NBUF, NCHUNK):
        hw[g].wait()


@jax.jit
def kernel(x, table):
    mesh = plsc.VectorSubcoreMesh(core_axis_name="c", subcore_axis_name="s")
    f = functools.partial(
        pl.kernel,
        out_type=jax.ShapeDtypeStruct((BATCH, SEQ, DIM), jnp.float32),
        mesh=mesh,
        scratch_types=(
            [pltpu.VMEM((RPW,), jnp.int32)]
            + [pltpu.VMEM((CHUNK, DIM), jnp.float32)] * NBUF
            + [pltpu.SemaphoreType.DMA] * (2 * NBUF)
        ),
    )(_sc_body)
    return f(x, table)


# speedup vs baseline: 1.0476x; 1.0003x over previous
import functools
import math

import jax
import jax.numpy as jnp
from jax import lax
from jax.experimental import pallas as pl
from jax.experimental.pallas import tpu as pltpu
from jax.experimental.pallas import tpu_sc as plsc

BATCH = 4
SEQ = 4096
DIM = 1024
NUM_ROWS = BATCH * SEQ
NC, NS, LANES = 2, 16, 16
NW = NC * NS
RPW = NUM_ROWS // NW
WPB = SEQ // RPW
CHUNK = 16
NCHUNK = RPW // CHUNK
NBUF = 7
LEAD = 6
SCALE = math.sqrt(DIM)


def _sc_body(x_hbm, table_hbm, out_hbm, idx_v,
             b0, b1, b2, b3, b4, b5, b6,
             sg0, sg1, sg2, sg3, sg4, sg5, sg6,
             so0, so1, so2, so3, so4, so5, so6):
    bufs = (b0, b1, b2, b3, b4, b5, b6)
    sgs = (sg0, sg1, sg2, sg3, sg4, sg5, sg6)
    sos = (so0, so1, so2, so3, so4, so5, so6)
    wid = lax.axis_index("s") * NC + lax.axis_index("c")
    batch = wid // WPB
    col0 = (wid % WPB) * RPW

    def gather(g):
        k = g % NBUF
        return pltpu.async_copy(
            table_hbm.at[idx_v.at[pl.ds(g * CHUNK, CHUNK)]], bufs[k], sgs[k])

    def writeback(g):
        k = g % NBUF
        return pltpu.async_copy(
            bufs[k], out_hbm.at[batch, pl.ds(col0 + g * CHUNK, CHUNK)], sos[k])

    def scale(k):
        def half_body(i, c2):
            r = i >> 1
            cb = (i & 1) * (DIM // 2)
            for c in range(DIM // LANES // 2):
                sl = pl.ds(cb + c * LANES, LANES)
                bufs[k][r, sl] = bufs[k][r, sl] * SCALE
            return c2
        lax.fori_loop(0, 2 * CHUNK, half_body, 0, unroll=False)

    head = 128
    pltpu.sync_copy(x_hbm.at[batch, pl.ds(col0, head)], idx_v.at[pl.ds(0, head)])
    hg = {g: gather(g) for g in range(LEAD)}
    pltpu.sync_copy(x_hbm.at[batch, pl.ds(col0 + head, RPW - head)],
                    idx_v.at[pl.ds(head, RPW - head)])
    hw = {}
    for g in range(NCHUNK):
        k = g % NBUF
        hg[g].wait()
        scale(k)
        hw[g] = writeback(g)
        if g + LEAD < NCHUNK:
            prev = g + LEAD - NBUF
            if prev >= 0:
                hw[prev].wait()
            hg[g + LEAD] = gather(g + LEAD)
    for g in range(NCHUNK - NBUF, NCHUNK):
        hw[g].wait()


@jax.jit
def kernel(x, table):
    mesh = plsc.VectorSubcoreMesh(core_axis_name="c", subcore_axis_name="s")
    f = functools.partial(
        pl.kernel,
        out_type=jax.ShapeDtypeStruct((BATCH, SEQ, DIM), jnp.float32),
        mesh=mesh,
        scratch_types=(
            [pltpu.VMEM((RPW,), jnp.int32)]
            + [pltpu.VMEM((CHUNK, DIM), jnp.float32)] * NBUF
            + [pltpu.SemaphoreType.DMA] * (2 * NBUF)
        ),
    )(_sc_body)
    return f(x, table)
